# Initial kernel scaffold; baseline (speedup 1.0000x reference)
import jax
import jax.numpy as jnp
from jax.experimental import pallas as pl


def kernel(x, edge_index, edge_timestamp, W, b):
    raise NotImplementedError("write your pallas kernel here")


# speedup vs baseline: 1.2185x; 1.2185x over previous
import functools

import jax
import jax.numpy as jnp
from jax import lax
from jax.experimental import pallas as pl
from jax.experimental.pallas import tpu as pltpu
from jax.experimental.pallas import tpu_sc as plsc

N_NODES = 10000
N_EDGES = 320000
D = 128
NW = 32
ROWS_PER_W = 320
N_PAD = NW * ROWS_PER_W
EC = 4000
N_CHUNKS = N_EDGES // EC
N_GROUPS = EC // 16
KSEG = D // 16

_mesh = plsc.VectorSubcoreMesh(core_axis_name="c", subcore_axis_name="s")


@functools.partial(
    pl.kernel,
    out_type=jax.ShapeDtypeStruct((N_PAD, D), jnp.float32),
    mesh=_mesh,
    scratch_types=[
        pltpu.VMEM((EC,), jnp.int32),
        pltpu.VMEM((EC,), jnp.int32),
        pltpu.VMEM((EC,), jnp.float32),
        pltpu.VMEM((EC + 16,), jnp.int32),
        pltpu.VMEM((EC + 16,), jnp.int32),
        pltpu.VMEM((EC + 16,), jnp.float32),
        pltpu.VMEM((16, D), jnp.float32),
        pltpu.VMEM((ROWS_PER_W + 1, D), jnp.float32),
        pltpu.SemaphoreType.DMA,
    ],
    compiler_params=pltpu.CompilerParams(needs_layout_passes=False),
)
def _agg_kernel(x_hbm, src_hbm, dst_hbm, ts_hbm, out_hbm,
                dstv, srcv, tsv, msrc, mdl, mts, rows, agg, sem):
    wid = lax.axis_index("s") * 2 + lax.axis_index("c")
    lo = wid * ROWS_PER_W
    hi = lo + ROWS_PER_W
    lov = jnp.full((16,), lo, dtype=jnp.int32)
    hiv = jnp.full((16,), hi, dtype=jnp.int32)

    neg_inf = jnp.full((16,), -jnp.inf, dtype=jnp.float32)

    def init_body(r, carry):
        for k in range(KSEG):
            agg[r, pl.ds(k * 16, 16)] = neg_inf
        return carry

    lax.fori_loop(0, ROWS_PER_W + 1, init_body, 0)

    def chunk_body(c, carry):
        base = c * EC
        pltpu.sync_copy(dst_hbm.at[pl.ds(base, EC)], dstv)
        pltpu.sync_copy(src_hbm.at[pl.ds(base, EC)], srcv)
        pltpu.sync_copy(ts_hbm.at[pl.ds(base, EC)], tsv)

        def scan_body(g, cnt):
            d = dstv[pl.ds(g * 16, 16)]
            m = (d >= lov) & (d < hiv)
            csum = plsc.cumsum(m.astype(jnp.int32))
            nm = csum[15]

            def do_pack(cnt_in):
                s = srcv[pl.ds(g * 16, 16)]
                t = tsv[pl.ds(g * 16, 16)]
                pos = cnt_in + csum - 1
                plsc.store_scatter(msrc, [pos], s, mask=m)
                plsc.store_scatter(mdl, [pos], d - lo, mask=m)
                plsc.store_scatter(mts, [pos], t, mask=m)
                return cnt_in + nm

            return lax.cond(nm > 0, do_pack, lambda cnt_in: cnt_in, cnt)

        mcnt = lax.fori_loop(0, N_GROUPS, scan_body, 0)

        padpos = mcnt + lax.iota(jnp.int32, 16)
        plsc.store_scatter(msrc, [padpos], jnp.zeros((16,), jnp.int32))
        plsc.store_scatter(mdl, [padpos],
                           jnp.full((16,), ROWS_PER_W, dtype=jnp.int32))
        plsc.store_scatter(mts, [padpos], jnp.zeros((16,), jnp.float32))
        ngroups = (mcnt + 15) // 16

        def grp_body(g, carry2):
            idxv = msrc[pl.ds(g * 16, 16)]
            pltpu.async_copy(x_hbm.at[idxv], rows, sem).wait()
            dlv = mdl[pl.ds(g * 16, 16)]
            tv = mts[pl.ds(g * 16, 16)]
            for j in range(16):
                dl = dlv[j]
                t = tv[j]
                for k in range(KSEG):
                    seg = rows[j, pl.ds(k * 16, 16)] * t
                    cur = agg[dl, pl.ds(k * 16, 16)]
                    agg[dl, pl.ds(k * 16, 16)] = jnp.maximum(cur, seg)
            return carry2

        lax.fori_loop(0, ngroups, grp_body, 0)
        return carry

    lax.fori_loop(0, N_CHUNKS, chunk_body, 0)

    pltpu.sync_copy(agg.at[pl.ds(0, ROWS_PER_W)],
                    out_hbm.at[pl.ds(lo, ROWS_PER_W)])


def _linear_body(agg_ref, w_ref, b_ref, o_ref):
    a = agg_ref[...]
    a = jnp.where(jnp.isfinite(a), a, 0.0)
    o_ref[...] = lax.dot_general(
        a, w_ref[...], (((1,), (1,)), ((), ())),
        preferred_element_type=jnp.float32) + b_ref[...]


_ROW_BLK = 400


def _linear_call(agg, W, b):
    return pl.pallas_call(
        _linear_body,
        grid=(N_NODES // _ROW_BLK,),
        in_specs=[
            pl.BlockSpec((_ROW_BLK, D), lambda i: (i, 0)),
            pl.BlockSpec((D, D), lambda i: (0, 0)),
            pl.BlockSpec((1, D), lambda i: (0, 0)),
        ],
        out_specs=pl.BlockSpec((_ROW_BLK, D), lambda i: (i, 0)),
        out_shape=jax.ShapeDtypeStruct((N_NODES, D), jnp.float32),
    )(agg, W, b.reshape(1, D))


def kernel(x, edge_index, edge_timestamp, W, b):
    src = edge_index[0]
    dst = edge_index[1]
    agg_pad = _agg_kernel(x, src, dst, edge_timestamp)
    return _linear_call(agg_pad[:N_NODES], W, b)
